# Initial kernel scaffold; baseline (speedup 1.0000x reference)
import jax
import jax.numpy as jnp
from jax.experimental import pallas as pl


def kernel(x, edge_index, batch, params):
    raise NotImplementedError("write your pallas kernel here")


# speedup vs baseline: 15.3445x; 15.3445x over previous
import functools
import math

import jax
import jax.numpy as jnp
from jax import lax
from jax.experimental import pallas as pl
from jax.experimental.pallas import tpu as pltpu
from jax.experimental.pallas import tpu_sc as plsc

N = 10000
E = 320000
D = 128
HID = 128
OUT = 8
HEADS = 4
S1 = 75
B = 64
C = S1 * HEADS
HD = HID // HEADS

NPAD = 10240
RB = 1024
NBLK = NPAD // RB
BP = B + 8
SPT = 79
EPT2 = SPT * 128
EPAD = 32 * EPT2
EPT1 = E // 32
ROWS_PER_TILE = NPAD // 16

_SCALE = 1.0 / math.sqrt(HID)


def _ln(x, g, b):
    mu = jnp.mean(x, axis=-1, keepdims=True)
    var = jnp.mean((x - mu) ** 2, axis=-1, keepdims=True)
    return (x - mu) / jnp.sqrt(var + 1e-5) * g + b


def _k_h(x_ref, w_ref, b_ref, o_ref):
    o_ref[...] = jnp.dot(x_ref[...], w_ref[...],
                         preferred_element_type=jnp.float32) + b_ref[...]


def _tc_h(x, w1, b1):
    return pl.pallas_call(
        _k_h,
        grid=(NBLK,),
        in_specs=[
            pl.BlockSpec((RB, D), lambda i: (i, 0)),
            pl.BlockSpec((D, HID), lambda i: (0, 0)),
            pl.BlockSpec((1, HID), lambda i: (0, 0)),
        ],
        out_specs=pl.BlockSpec((RB, HID), lambda i: (i, 0)),
        out_shape=jax.ShapeDtypeStruct((NPAD, HID), jnp.float32),
    )(x, w1, b1)


def _k_prep(s1_ref, wq1_ref, bq1_ref, gkw_ref, gkb_ref, s3_ref, wq3_ref, bq3_ref,
            qp1_ref, wl_ref, bl_ref, qp3_ref):
    qp1 = jnp.dot(s1_ref[...], wq1_ref[...],
                  preferred_element_type=jnp.float32) + bq1_ref[...]
    qp1_ref[...] = qp1
    gkw = gkw_ref[...]
    gkb = gkb_ref[...]
    wl_parts = []
    bl_parts = []
    for h in range(HEADS):
        qh = qp1[:, h * HD:(h + 1) * HD]
        wh = gkw[:, h * HD:(h + 1) * HD]
        bh = gkb[:, h * HD:(h + 1) * HD]
        wl_parts.append(jax.lax.dot_general(
            wh, qh, (((1,), (1,)), ((), ())),
            preferred_element_type=jnp.float32))
        bl_parts.append(jax.lax.dot_general(
            bh, qh, (((1,), (1,)), ((), ())),
            preferred_element_type=jnp.float32))
    wl_ref[...] = jnp.concatenate(wl_parts, axis=1) * _SCALE
    bl_ref[...] = jnp.concatenate(bl_parts, axis=1) * _SCALE
    qp3_ref[...] = jnp.dot(s3_ref[...], wq3_ref[...],
                           preferred_element_type=jnp.float32) + bq3_ref[...]


def _tc_prep(s1, wq1, bq1, gkw, gkb, s3, wq3, bq3):
    full = lambda shp: pl.BlockSpec(shp, lambda: tuple(0 for _ in shp))
    return pl.pallas_call(
        _k_prep,
        grid=(),
        in_specs=[full((S1, HID)), full((HID, HID)), full((1, HID)),
                  full((HID, HID)), full((1, HID)),
                  full((1, HID)), full((HID, HID)), full((1, HID))],
        out_specs=[full((S1, HID)), full((HID, C)), full((1, C)), full((1, HID))],
        out_shape=[jax.ShapeDtypeStruct((S1, HID), jnp.float32),
                   jax.ShapeDtypeStruct((HID, C), jnp.float32),
                   jax.ShapeDtypeStruct((1, C), jnp.float32),
                   jax.ShapeDtypeStruct((1, HID), jnp.float32)],
    )(s1, wq1, bq1, gkw, gkb, s3, wq3, bq3)


def _k_dis(degp_ref, h_ref, dis_ref, hd_ref):
    deg = jnp.sum(degp_ref[...], axis=0, keepdims=True) + 1.0
    dis = lax.rsqrt(deg)
    disc = dis.reshape(RB, 1)
    dis_ref[...] = disc
    hd_ref[...] = h_ref[...] * disc


def _tc_dis(degp, h):
    return pl.pallas_call(
        _k_dis,
        grid=(NBLK,),
        in_specs=[
            pl.BlockSpec((32, RB), lambda i: (0, i)),
            pl.BlockSpec((RB, HID), lambda i: (i, 0)),
        ],
        out_specs=[pl.BlockSpec((RB, 1), lambda i: (i, 0)),
                   pl.BlockSpec((RB, HID), lambda i: (i, 0))],
        out_shape=[jax.ShapeDtypeStruct((NPAD, 1), jnp.float32),
                   jax.ShapeDtypeStruct((NPAD, HID), jnp.float32)],
    )(degp, h)


def _k_aggvl(a0_ref, a1_ref, h_ref, dis_ref, gvw_ref, gvb_ref, wl_ref, bl_ref,
             v_ref, l_ref):
    dis = dis_ref[...]
    agg = dis * (a0_ref[0] + a1_ref[0]) + h_ref[...] * (dis * dis)
    v_ref[...] = jnp.dot(agg, gvw_ref[...],
                         preferred_element_type=jnp.float32) + gvb_ref[...]
    l_ref[...] = jnp.dot(agg, wl_ref[...],
                         preferred_element_type=jnp.float32) + bl_ref[...]


def _tc_aggvl(acc, h, dis, gvw, gvb, wl, bl):
    return pl.pallas_call(
        _k_aggvl,
        grid=(NBLK,),
        in_specs=[
            pl.BlockSpec((1, RB, HID), lambda i: (0, i, 0)),
            pl.BlockSpec((1, RB, HID), lambda i: (1, i, 0)),
            pl.BlockSpec((RB, HID), lambda i: (i, 0)),
            pl.BlockSpec((RB, 1), lambda i: (i, 0)),
            pl.BlockSpec((HID, HID), lambda i: (0, 0)),
            pl.BlockSpec((1, HID), lambda i: (0, 0)),
            pl.BlockSpec((HID, C), lambda i: (0, 0)),
            pl.BlockSpec((1, C), lambda i: (0, 0)),
        ],
        out_specs=[pl.BlockSpec((RB, HID), lambda i: (i, 0)),
                   pl.BlockSpec((RB, C), lambda i: (i, 0))],
        out_shape=[jax.ShapeDtypeStruct((NPAD, HID), jnp.float32),
                   jax.ShapeDtypeStruct((NPAD, C), jnp.float32)],
    )(acc, acc, h, dis, gvw, gvb, wl, bl)


def _k_segmax(l_ref, b_ref, m_ref, ms):
    i = pl.program_id(0)

    @pl.when(i == 0)
    def _():
        ms[...] = jnp.full((BP, C), -1e30, jnp.float32)

    lb = l_ref[...]
    bcol = b_ref[0]
    bmin = b_ref[0, 0, 0]
    bmax = b_ref[0, RB - 1, 0]

    def body(b, carry):
        mask = bcol == b
        colmax = jnp.max(jnp.where(mask, lb, -1e30), axis=0, keepdims=True)
        ms[pl.ds(b, 1), :] = jnp.maximum(ms[pl.ds(b, 1), :], colmax)
        return carry

    lax.fori_loop(bmin, bmax + 1, body, 0)

    @pl.when(i == NBLK - 1)
    def _():
        m_ref[...] = ms[0:B, :]


def _tc_segmax(l, batchr):
    return pl.pallas_call(
        _k_segmax,
        grid=(NBLK,),
        in_specs=[
            pl.BlockSpec((RB, C), lambda i: (i, 0)),
            pl.BlockSpec((1, RB, 1), lambda i: (i, 0, 0)),
        ],
        out_specs=pl.BlockSpec((B, C), lambda i: (0, 0)),
        out_shape=jax.ShapeDtypeStruct((B, C), jnp.float32),
        scratch_shapes=[pltpu.VMEM((BP, C), jnp.float32)],
    )(l, batchr)


def _k_segsum(l_ref, v_ref, b_ref, m_ref, d_ref, o_ref, ds, os):
    i = pl.program_id(0)

    @pl.when(i == 0)
    def _():
        ds[...] = jnp.zeros((B, C), jnp.float32)
        os[...] = jnp.zeros((BP, S1, HID), jnp.float32)

    bcol = b_ref[0]
    onehot = (bcol ==
              lax.broadcasted_iota(jnp.int32, (RB, B), 1)).astype(jnp.float32)
    mrows = jnp.dot(onehot, m_ref[...], preferred_element_type=jnp.float32)
    w = jnp.exp(l_ref[...] - mrows)
    ds[...] += jax.lax.dot_general(onehot, w, (((0,), (0,)), ((), ())),
                                   preferred_element_type=jnp.float32)
    vb = v_ref[...]
    bmin = b_ref[0, 0, 0]
    bmax = b_ref[0, RB - 1, 0]

    def body(b, carry):
        maskf = (bcol == b).astype(jnp.float32)
        wb = w * maskf
        for h in range(HEADS):
            oh = jax.lax.dot_general(
                wb[:, h * S1:(h + 1) * S1], vb[:, h * HD:(h + 1) * HD],
                (((0,), (0,)), ((), ())),
                preferred_element_type=jnp.float32)
            os[pl.ds(b, 1), :, pl.ds(h * HD, HD)] += oh[None]
        return carry

    lax.fori_loop(bmin, bmax + 1, body, 0)

    @pl.when(i == NBLK - 1)
    def _():
        dd = ds[...]
        d_ref[...] = jnp.where(dd == 0.0, 1.0, dd)
        o_ref[...] = os[0:B]


def _tc_segsum(l, v, batchr, m):
    return pl.pallas_call(
        _k_segsum,
        grid=(NBLK,),
        in_specs=[
            pl.BlockSpec((RB, C), lambda i: (i, 0)),
            pl.BlockSpec((RB, HID), lambda i: (i, 0)),
            pl.BlockSpec((1, RB, 1), lambda i: (i, 0, 0)),
            pl.BlockSpec((B, C), lambda i: (0, 0)),
        ],
        out_specs=[pl.BlockSpec((B, C), lambda i: (0, 0)),
                   pl.BlockSpec((B, S1, HID), lambda i: (0, 0, 0))],
        out_shape=[jax.ShapeDtypeStruct((B, C), jnp.float32),
                   jax.ShapeDtypeStruct((B, S1, HID), jnp.float32)],
        scratch_shapes=[pltpu.VMEM((B, C), jnp.float32),
                        pltpu.VMEM((BP, S1, HID), jnp.float32)],
    )(l, v, batchr, m)


def _mab_dense(q, k, v, wo, bo, g0, b0, g1, b1):
    parts = []
    for h in range(HEADS):
        qh = q[:, h * HD:(h + 1) * HD]
        kh = k[:, h * HD:(h + 1) * HD]
        vh = v[:, h * HD:(h + 1) * HD]
        sc = jax.lax.dot_general(qh, kh, (((1,), (1,)), ((), ())),
                                 preferred_element_type=jnp.float32) * _SCALE
        mx = jnp.max(sc, axis=-1, keepdims=True)
        e = jnp.exp(sc - mx)
        a = e / jnp.sum(e, axis=-1, keepdims=True)
        parts.append(qh + jnp.dot(a, vh, preferred_element_type=jnp.float32))
    o = jnp.concatenate(parts, axis=1)
    o = _ln(o, g0, b0)
    o = o + jax.nn.relu(jnp.dot(o, wo, preferred_element_type=jnp.float32) + bo)
    return _ln(o, g1, b1)


def _k_tail(o_ref, d_ref, qp1_ref, qp3_ref,
            wo1_ref, bo1_ref, g01_ref, b01_ref, g11_ref, b11_ref,
            wq2_ref, bq2_ref, wk2_ref, bk2_ref, wv2_ref, bv2_ref,
            wo2_ref, bo2_ref, g02_ref, b02_ref, g12_ref, b12_ref,
            wk3_ref, bk3_ref, wv3_ref, bv3_ref,
            wo3_ref, bo3_ref, g03_ref, b03_ref, g13_ref, b13_ref,
            w2_ref, b2_ref, y_ref):
    d4 = d_ref[0]
    sel = (lax.broadcasted_iota(jnp.int32, (HEADS, HID), 1) // HD ==
           lax.broadcasted_iota(jnp.int32, (HEADS, HID), 0)).astype(jnp.float32)
    div = jax.lax.dot_general(d4, sel, (((0,), (0,)), ((), ())),
                              preferred_element_type=jnp.float32)
    o = qp1_ref[...] + o_ref[0] / div
    o = _ln(o, g01_ref[...], b01_ref[...])
    o = o + jax.nn.relu(jnp.dot(o, wo1_ref[...],
                                preferred_element_type=jnp.float32) + bo1_ref[...])
    out1 = _ln(o, g11_ref[...], b11_ref[...])

    qp2 = jnp.dot(out1, wq2_ref[...], preferred_element_type=jnp.float32) + bq2_ref[...]
    k2 = jnp.dot(out1, wk2_ref[...], preferred_element_type=jnp.float32) + bk2_ref[...]
    v2 = jnp.dot(out1, wv2_ref[...], preferred_element_type=jnp.float32) + bv2_ref[...]
    out2 = _mab_dense(qp2, k2, v2, wo2_ref[...], bo2_ref[...],
                      g02_ref[...], b02_ref[...], g12_ref[...], b12_ref[...])

    k3 = jnp.dot(out2, wk3_ref[...], preferred_element_type=jnp.float32) + bk3_ref[...]
    v3 = jnp.dot(out2, wv3_ref[...], preferred_element_type=jnp.float32) + bv3_ref[...]
    out3 = _mab_dense(qp3_ref[...], k3, v3, wo3_ref[...], bo3_ref[...],
                      g03_ref[...], b03_ref[...], g13_ref[...], b13_ref[...])
    y_ref[0] = jnp.dot(out3, w2_ref[...],
                       preferred_element_type=jnp.float32) + b2_ref[...]


def _tc_tail(o, d4, qp1, qp3, pv):
    vec = lambda: pl.BlockSpec((1, HID), lambda i: (0, 0))
    mat = lambda: pl.BlockSpec((HID, HID), lambda i: (0, 0))
    return pl.pallas_call(
        _k_tail,
        grid=(B,),
        in_specs=[
            pl.BlockSpec((1, S1, HID), lambda i: (i, 0, 0)),
            pl.BlockSpec((1, HEADS, S1), lambda i: (i, 0, 0)),
            pl.BlockSpec((S1, HID), lambda i: (0, 0)),
            vec(),
            mat(), vec(), vec(), vec(), vec(), vec(),
            mat(), vec(), mat(), vec(), mat(), vec(),
            mat(), vec(), vec(), vec(), vec(), vec(),
            mat(), vec(), mat(), vec(),
            mat(), vec(), vec(), vec(), vec(), vec(),
            pl.BlockSpec((HID, OUT), lambda i: (0, 0)),
            pl.BlockSpec((1, OUT), lambda i: (0, 0)),
        ],
        out_specs=pl.BlockSpec((1, 1, OUT), lambda i: (i, 0, 0)),
        out_shape=jax.ShapeDtypeStruct((B, 1, OUT), jnp.float32),
    )(o, d4, qp1, qp3, *pv)


def _sc_deg(dst, zflat):
    mesh = plsc.VectorSubcoreMesh(core_axis_name="c", subcore_axis_name="s")

    @functools.partial(
        pl.kernel, mesh=mesh,
        compiler_params=pltpu.CompilerParams(needs_layout_passes=False),
        out_type=jax.ShapeDtypeStruct((32, NPAD), jnp.float32),
        scratch_types=[pltpu.VMEM((EPT1,), jnp.int32),
                       pltpu.VMEM((NPAD,), jnp.float32)],
    )
    def k(dst_hbm, zf_hbm, out_hbm, idx_v, deg_v):
        cid = lax.axis_index("c")
        sid = lax.axis_index("s")
        wid = sid * 2 + cid
        pltpu.sync_copy(zf_hbm, deg_v)
        pltpu.sync_copy(dst_hbm.at[pl.ds(wid * EPT1, EPT1)], idx_v)
        ones = jnp.ones((16,), jnp.float32)

        def body(j, carry):
            idx16 = idx_v[pl.ds(j * 16, 16)]
            plsc.addupdate_scatter(deg_v, [idx16], ones)
            return carry

        lax.fori_loop(0, EPT1 // 16, body, 0)
        pltpu.sync_copy(deg_v, out_hbm.at[wid])

    return k(dst, zflat)


def _sc_agg(hd_pad, srcp, dstp, zrows):
    mesh = plsc.VectorSubcoreMesh(core_axis_name="c", subcore_axis_name="s")

    @functools.partial(
        pl.kernel, mesh=mesh,
        compiler_params=pltpu.CompilerParams(needs_layout_passes=False),
        out_type=jax.ShapeDtypeStruct((2 * NPAD, HID), jnp.float32),
        scratch_types=[pltpu.VMEM((SPT, 128), jnp.int32),
                       pltpu.VMEM((SPT, 128), jnp.int32),
                       pltpu.VMEM((128, HID), jnp.float32),
                       pltpu.VMEM_SHARED((NPAD, HID), jnp.float32),
                       pltpu.SemaphoreType.DMA],
    )
    def k(hd_hbm, srcp_hbm, dstp_hbm, zz_hbm, out_hbm, src_v, dst_v, rows_v,
          acc_sh, sem):
        cid = lax.axis_index("c")
        sid = lax.axis_index("s")
        wid = sid * 2 + cid
        pltpu.sync_copy(zz_hbm, acc_sh.at[pl.ds(sid * ROWS_PER_TILE,
                                                ROWS_PER_TILE)])
        pltpu.sync_copy(srcp_hbm.at[wid], src_v)
        pltpu.sync_copy(dstp_hbm.at[wid], dst_v)
        plsc.subcore_barrier()

        def body(j, carry):
            pltpu.async_copy(hd_hbm.at[src_v.at[j]], rows_v, sem).wait()
            pltpu.sync_copy(rows_v, acc_sh.at[dst_v.at[j]], add=True)
            return carry

        lax.fori_loop(0, SPT, body, 0)
        plsc.subcore_barrier()
        pltpu.sync_copy(
            acc_sh.at[pl.ds(sid * ROWS_PER_TILE, ROWS_PER_TILE)],
            out_hbm.at[pl.ds(cid * NPAD + sid * ROWS_PER_TILE, ROWS_PER_TILE)])

    return k(hd_pad, srcp, dstp, zrows)


def kernel(x, edge_index, batch, params):
    p = params
    src = edge_index[0]
    dst = edge_index[1]

    zrows = jnp.zeros((ROWS_PER_TILE, HID), jnp.float32)
    zflat = jnp.zeros((NPAD,), jnp.float32)
    x_pad = jnp.concatenate([x, jnp.zeros((NPAD - N, D), jnp.float32)])
    batch_pad = jnp.concatenate([batch, jnp.full((NPAD - N,), B, jnp.int32)])
    srcp = jnp.concatenate(
        [src, jnp.full((EPAD - E,), N, jnp.int32)]).reshape(32, SPT, 128)
    dstp = jnp.concatenate(
        [dst, jnp.full((EPAD - E,), N, jnp.int32)]).reshape(32, SPT, 128)
    batchr = batch_pad.reshape(NBLK, RB, 1)

    r2 = lambda v: v.reshape(1, -1)

    h = _tc_h(x_pad, p['lin1_W'], r2(p['lin1_b']))
    qp1, wl, bl, qp3 = _tc_prep(p['S1'][0], p['Wq1'], r2(p['bq1']),
                                p['gk_W'], r2(p['gk_b']),
                                p['S3'][0], p['Wq3'], r2(p['bq3']))
    degp = _sc_deg(dst, zflat)
    dis, hd = _tc_dis(degp, h)
    accf = _sc_agg(hd, srcp, dstp, zrows)
    acc = accf.reshape(2, NPAD, HID)
    v, l = _tc_aggvl(acc, h, dis, p['gv_W'], r2(p['gv_b']), wl, bl)
    m = _tc_segmax(l, batchr)
    d, o = _tc_segsum(l, v, batchr, m)
    d4 = d.reshape(B, HEADS, S1)
    pv = (p['Wo1'], r2(p['bo1']), r2(p['g0_1']), r2(p['b0_1']),
          r2(p['g1_1']), r2(p['b1_1']),
          p['Wq2'], r2(p['bq2']), p['Wk2'], r2(p['bk2']),
          p['Wv2'], r2(p['bv2']),
          p['Wo2'], r2(p['bo2']), r2(p['g0_2']), r2(p['b0_2']),
          r2(p['g1_2']), r2(p['b1_2']),
          p['Wk3'], r2(p['bk3']), p['Wv3'], r2(p['bv3']),
          p['Wo3'], r2(p['bo3']), r2(p['g0_3']), r2(p['b0_3']),
          r2(p['g1_3']), r2(p['b1_3']),
          p['lin2_W'], r2(p['lin2_b']))
    y = _tc_tail(o, d4, qp1, qp3, pv)
    return y.reshape(B, OUT)
